# Initial kernel scaffold; baseline (speedup 1.0000x reference)
import jax
import jax.numpy as jnp
from jax.experimental import pallas as pl


def kernel(node_features, edge_index, edge_attr, W_in, b_in, We1, be1, We2, be2, Wc0, bc0, Wc1, bc1, Wc2, bc2, Wo, bo):
    raise NotImplementedError("write your pallas kernel here")


# speedup vs baseline: 6.0927x; 6.0927x over previous
import functools

import jax
import jax.numpy as jnp
from jax import lax
from jax.experimental import pallas as pl
from jax.experimental.pallas import tpu as pltpu
from jax.experimental.pallas import tpu_sc as plsc

N = 10000
IN_DIM = 128
EDGE_DIM = 16
HID = 128

NW = 32
NP = 10240
E = 320000
CHUNK = 128
N_CHUNKS = 79
PER_TILE = CHUNK * N_CHUNKS
EPAD = PER_TILE * NW
ROWS_PER_TILE = NP // 16
EB = 1024


def _edge_mlp_body(eaT_ref, W1T_ref, b1_ref, W2T_ref, b2_ref, out_ref):
    t = jnp.dot(W1T_ref[...], eaT_ref[...], preferred_element_type=jnp.float32)
    t = jnp.maximum(t + b1_ref[...], 0.0)
    s = jnp.dot(W2T_ref[...], t, preferred_element_type=jnp.float32) + b2_ref[...]
    out_ref[...] = jax.nn.sigmoid(s)


def _t0_body(x_ref, Win_ref, bin_ref, Wc0_ref, degb_ref, h0_ref, g0_ref, dis_ref):
    d = lax.rsqrt(degb_ref[0] + degb_ref[1] + 1.0)
    h0 = jnp.dot(x_ref[...], Win_ref[...], preferred_element_type=jnp.float32)
    h0 = jnp.maximum(h0 + bin_ref[...], 0.0)
    h0_ref[...] = h0
    dis_ref[...] = d
    g0_ref[...] = jnp.dot(h0, Wc0_ref[...], preferred_element_type=jnp.float32) * d


def _layer_body(h_ref, p_ref, g_ref, dis_ref, b_ref, W_ref, hn_ref, gn_ref):
    dis = dis_ref[...]
    hn = dis * (p_ref[0] + p_ref[1] + g_ref[...]) + b_ref[...] + h_ref[...]
    hn = jnp.maximum(hn, 0.0)
    hn_ref[...] = hn
    gn_ref[...] = jnp.dot(hn, W_ref[...], preferred_element_type=jnp.float32) * dis


def _final_body(h_ref, p_ref, g_ref, dis_ref, b_ref, Wo_ref, bo_ref, out_ref):
    dis = dis_ref[...]
    hn = dis * (p_ref[0] + p_ref[1] + g_ref[...]) + b_ref[...] + h_ref[...]
    hn = jnp.maximum(hn, 0.0)
    out_ref[...] = jnp.dot(hn, Wo_ref[...], preferred_element_type=jnp.float32) + bo_ref[...]


_ROWB = 256
_GRID = NP // _ROWB

_row_spec = pl.BlockSpec((_ROWB, HID), lambda i: (i, 0))
_w_spec = pl.BlockSpec((HID, HID), lambda i: (0, 0))
_b_spec = pl.BlockSpec((1, HID), lambda i: (0, 0))
_p_spec = pl.BlockSpec((2, _ROWB, HID), lambda i: (0, i, 0))
_f32 = jnp.float32
_node_sds = jax.ShapeDtypeStruct((NP, HID), _f32)


def _zero_acc_slice(rows_v, acc_sh, s):
    zeros16 = jnp.zeros((16,), _f32)

    def zr(i, _):
        for j in range(8):
            rows_v[i, pl.ds(j * 16, 16)] = zeros16
        return 0
    lax.fori_loop(0, CHUNK, zr, 0)

    row0 = s * ROWS_PER_TILE
    for k in range(ROWS_PER_TILE // CHUNK):
        pltpu.sync_copy(rows_v, acc_sh.at[pl.ds(row0 + k * CHUNK, CHUNK)])


def _deg_body(dst_hbm, ew_hbm, out_hbm, acc_sh, dst_v, ew_v, rows_v):
    c = lax.axis_index("c")
    s = lax.axis_index("s")
    w = c * 16 + s
    _zero_acc_slice(rows_v, acc_sh, s)
    plsc.subcore_barrier()

    base = w * PER_TILE

    def chunk_body(k, _):
        e0 = base + k * CHUNK
        pltpu.sync_copy(dst_hbm.at[pl.ds(e0, CHUNK)], dst_v)
        pltpu.sync_copy(ew_hbm.at[pl.ds(e0, CHUNK)], ew_v)

        def r_body(i, _):
            wv = ew_v[pl.ds(i * 16, 16)]
            for l in range(16):
                wb = jnp.full((16,), wv[l], dtype=_f32)
                r = i * 16 + l
                for j in range(8):
                    rows_v[r, pl.ds(j * 16, 16)] = wb
            return 0
        lax.fori_loop(0, CHUNK // 16, r_body, 0)

        pltpu.sync_copy(rows_v, acc_sh.at[dst_v], add=True)
        return 0
    lax.fori_loop(0, N_CHUNKS, chunk_body, 0)

    plsc.subcore_barrier()
    row0 = s * ROWS_PER_TILE
    pltpu.sync_copy(acc_sh.at[pl.ds(row0, ROWS_PER_TILE)],
                    out_hbm.at[c, pl.ds(row0, ROWS_PER_TILE)])


def _agg_body(g_hbm, src_hbm, dst_hbm, ew_hbm, out_hbm,
              acc_sh, src_v, dst_v, ew_v, rows_v, sem):
    c = lax.axis_index("c")
    s = lax.axis_index("s")
    w = c * 16 + s
    _zero_acc_slice(rows_v, acc_sh, s)
    plsc.subcore_barrier()

    base = w * PER_TILE

    def chunk_body(k, _):
        e0 = base + k * CHUNK
        pltpu.sync_copy(src_hbm.at[pl.ds(e0, CHUNK)], src_v)
        pltpu.sync_copy(dst_hbm.at[pl.ds(e0, CHUNK)], dst_v)
        pltpu.sync_copy(ew_hbm.at[pl.ds(e0, CHUNK)], ew_v)
        pltpu.async_copy(g_hbm.at[src_v], rows_v, sem).wait()

        def r_body(i, _):
            wv = ew_v[pl.ds(i * 16, 16)]
            for l in range(16):
                wb = jnp.full((16,), wv[l], dtype=_f32)
                r = i * 16 + l
                for j in range(8):
                    sl = pl.ds(j * 16, 16)
                    rows_v[r, sl] = rows_v[r, sl] * wb
            return 0
        lax.fori_loop(0, CHUNK // 16, r_body, 0)

        pltpu.sync_copy(rows_v, acc_sh.at[dst_v], add=True)
        return 0
    lax.fori_loop(0, N_CHUNKS, chunk_body, 0)

    plsc.subcore_barrier()
    row0 = s * ROWS_PER_TILE
    pltpu.sync_copy(acc_sh.at[pl.ds(row0, ROWS_PER_TILE)],
                    out_hbm.at[c, pl.ds(row0, ROWS_PER_TILE)])


@functools.cache
def _sc_kernels():
    mesh = plsc.VectorSubcoreMesh(core_axis_name="c", subcore_axis_name="s")
    deg_k = pl.kernel(
        _deg_body,
        out_type=jax.ShapeDtypeStruct((2, NP, HID), _f32),
        mesh=mesh,
        scratch_types=[
            pltpu.VMEM_SHARED((NP, HID), _f32),
            pltpu.VMEM((CHUNK,), jnp.int32),
            pltpu.VMEM((CHUNK,), _f32),
            pltpu.VMEM((CHUNK, HID), _f32),
        ],
    )
    agg_k = pl.kernel(
        _agg_body,
        out_type=jax.ShapeDtypeStruct((2, NP, HID), _f32),
        mesh=mesh,
        scratch_types=[
            pltpu.VMEM_SHARED((NP, HID), _f32),
            pltpu.VMEM((CHUNK,), jnp.int32),
            pltpu.VMEM((CHUNK,), jnp.int32),
            pltpu.VMEM((CHUNK,), _f32),
            pltpu.VMEM((CHUNK, HID), _f32),
            pltpu.SemaphoreType.DMA,
        ],
    )
    return deg_k, agg_k


def kernel(node_features, edge_index, edge_attr, W_in, b_in, We1, be1, We2, be2,
           Wc0, bc0, Wc1, bc1, Wc2, bc2, Wo, bo):
    f32 = jnp.float32
    x = jnp.concatenate(
        [node_features.astype(f32), jnp.zeros((NP - N, IN_DIM), f32)])
    pad_idx = jnp.full((EPAD - E,), NP - 1, dtype=jnp.int32)
    src = jnp.concatenate([edge_index[0], pad_idx])
    dst = jnp.concatenate([edge_index[1], pad_idx])
    eaT = jnp.concatenate(
        [edge_attr.astype(f32), jnp.zeros((EPAD - E, EDGE_DIM), f32)]).T

    ew2 = pl.pallas_call(
        _edge_mlp_body,
        grid=(EPAD // EB,),
        in_specs=[
            pl.BlockSpec((EDGE_DIM, EB), lambda i: (0, i)),
            pl.BlockSpec((96, EDGE_DIM), lambda i: (0, 0)),
            pl.BlockSpec((96, 1), lambda i: (0, 0)),
            pl.BlockSpec((1, 96), lambda i: (0, 0)),
            pl.BlockSpec((1, 1), lambda i: (0, 0)),
        ],
        out_specs=pl.BlockSpec((1, EB), lambda i: (0, i)),
        out_shape=jax.ShapeDtypeStruct((1, EPAD), f32),
    )(eaT, We1.T, be1[:, None], We2.T, be2[:, None])
    ew = ew2.reshape(EPAD)

    deg_k, agg_k = _sc_kernels()

    degb = deg_k(dst, ew)

    h0, g0, dis_b = pl.pallas_call(
        _t0_body,
        grid=(_GRID,),
        in_specs=[_row_spec, _w_spec, _b_spec, _w_spec, _p_spec],
        out_specs=[_row_spec, _row_spec, _row_spec],
        out_shape=[_node_sds, _node_sds, _node_sds],
    )(x, W_in, b_in[None, :], Wc0, degb)

    h, g = h0, g0
    for W_next, b_prev, last in ((Wc1, bc0, False), (Wc2, bc1, False), (Wo, bc2, True)):
        p = agg_k(g, src, dst, ew)
        if not last:
            h, g = pl.pallas_call(
                _layer_body,
                grid=(_GRID,),
                in_specs=[_row_spec, _p_spec, _row_spec, _row_spec, _b_spec, _w_spec],
                out_specs=[_row_spec, _row_spec],
                out_shape=[_node_sds, _node_sds],
            )(h, p, g, dis_b, b_prev[None, :], W_next)
        else:
            out = pl.pallas_call(
                _final_body,
                grid=(_GRID,),
                in_specs=[_row_spec, _p_spec, _row_spec, _row_spec, _b_spec,
                          _w_spec, _b_spec],
                out_specs=_row_spec,
                out_shape=_node_sds,
            )(h, p, g, dis_b, b_prev[None, :], W_next, bo[None, :])
    return out[:N]


# speedup vs baseline: 6.1668x; 1.0122x over previous
import functools

import jax
import jax.numpy as jnp
from jax import lax
from jax.experimental import pallas as pl
from jax.experimental.pallas import tpu as pltpu
from jax.experimental.pallas import tpu_sc as plsc

N = 10000
IN_DIM = 128
EDGE_DIM = 16
HID = 128

NW = 32
NP = 10240
E = 320000
CHUNK = 64
N_CHUNKS = 160
PER_TILE = CHUNK * N_CHUNKS
EPAD = PER_TILE * NW
ROWS_PER_TILE = NP // 16
EB = 1024


def _edge_mlp_body(eaT_ref, W1T_ref, b1_ref, W2T_ref, b2_ref, out_ref):
    t = jnp.dot(W1T_ref[...], eaT_ref[...], preferred_element_type=jnp.float32)
    t = jnp.maximum(t + b1_ref[...], 0.0)
    s = jnp.dot(W2T_ref[...], t, preferred_element_type=jnp.float32) + b2_ref[...]
    out_ref[...] = jax.nn.sigmoid(s)


def _t0_body(x_ref, Win_ref, bin_ref, Wc0_ref, degb_ref, h0_ref, g0_ref, dis_ref):
    d = lax.rsqrt(degb_ref[0] + degb_ref[1] + 1.0)
    h0 = jnp.dot(x_ref[...], Win_ref[...], preferred_element_type=jnp.float32)
    h0 = jnp.maximum(h0 + bin_ref[...], 0.0)
    h0_ref[...] = h0
    dis_ref[...] = d
    g0_ref[...] = jnp.dot(h0, Wc0_ref[...], preferred_element_type=jnp.float32) * d


def _layer_body(h_ref, p_ref, g_ref, dis_ref, b_ref, W_ref, hn_ref, gn_ref):
    dis = dis_ref[...]
    hn = dis * (p_ref[0] + p_ref[1] + g_ref[...]) + b_ref[...] + h_ref[...]
    hn = jnp.maximum(hn, 0.0)
    hn_ref[...] = hn
    gn_ref[...] = jnp.dot(hn, W_ref[...], preferred_element_type=jnp.float32) * dis


def _final_body(h_ref, p_ref, g_ref, dis_ref, b_ref, Wo_ref, bo_ref, out_ref):
    dis = dis_ref[...]
    hn = dis * (p_ref[0] + p_ref[1] + g_ref[...]) + b_ref[...] + h_ref[...]
    hn = jnp.maximum(hn, 0.0)
    out_ref[...] = jnp.dot(hn, Wo_ref[...], preferred_element_type=jnp.float32) + bo_ref[...]


_ROWB = 256
_GRID = NP // _ROWB

_row_spec = pl.BlockSpec((_ROWB, HID), lambda i: (i, 0))
_w_spec = pl.BlockSpec((HID, HID), lambda i: (0, 0))
_b_spec = pl.BlockSpec((1, HID), lambda i: (0, 0))
_p_spec = pl.BlockSpec((2, _ROWB, HID), lambda i: (0, i, 0))
_f32 = jnp.float32
_node_sds = jax.ShapeDtypeStruct((NP, HID), _f32)


def _deg_body(dst_hbm, ew_hbm, out_hbm, acc_sh,
              d0_v, d1_v, d2_v, d3_v, e0_v, e1_v, e2_v, e3_v, r0_v, r1_v,
              s0_s, s1_s, sd0, sd1, sd2, sd3, se0, se1, se2, se3):
    c = lax.axis_index("c")
    s = lax.axis_index("s")
    w = c * 16 + s
    rows = (r0_v, r1_v)
    dstb = (d0_v, d1_v, d2_v, d3_v)
    ewb = (e0_v, e1_v, e2_v, e3_v)
    sems = (s0_s, s1_s)
    semd = (sd0, sd1, sd2, sd3)
    seme = (se0, se1, se2, se3)

    pltpu.sync_copy(dst_hbm.at[w, 0], dstb[0])
    pltpu.sync_copy(ew_hbm.at[w, 0], ewb[0])
    pltpu.async_copy(dst_hbm.at[w, 1], dstb[1], semd[1])
    pltpu.async_copy(ew_hbm.at[w, 1], ewb[1], seme[1])

    zeros16 = jnp.zeros((16,), _f32)

    def zb(i, _):
        for j in range(8):
            sl = pl.ds(j * 16, 16)
            r0_v[i, sl] = zeros16
            r1_v[i, sl] = zeros16
        return 0
    lax.fori_loop(0, CHUNK, zb, 0)

    row0 = s * ROWS_PER_TILE
    for k in range(ROWS_PER_TILE // CHUNK):
        pltpu.sync_copy(r0_v, acc_sh.at[pl.ds(row0 + k * CHUNK, CHUNK)])
    plsc.subcore_barrier()

    for b in range(2):
        pltpu.async_copy(rows[b], acc_sh.at[dstb[0]], sems[b], add=True)

    def fill_scatter(k, b, q, wait_idx):
        q2 = (q + 2) % 4
        pltpu.make_async_copy(rows[b], acc_sh.at[dstb[0]], sems[b]).wait()
        pltpu.async_copy(dst_hbm.at[w, k + 2], dstb[q2], semd[q2])
        pltpu.async_copy(ew_hbm.at[w, k + 2], ewb[q2], seme[q2])
        if wait_idx:
            pltpu.make_async_copy(ew_hbm.at[w, k], ewb[q], seme[q]).wait()
            pltpu.make_async_copy(dst_hbm.at[w, k], dstb[q], semd[q]).wait()

        def r_body(i, _):
            wv = ewb[q][pl.ds(i * 16, 16)]
            for l in range(16):
                wb = jnp.full((16,), wv[l], dtype=_f32)
                r = i * 16 + l
                for j in range(8):
                    rows[b][r, pl.ds(j * 16, 16)] = wb
            return 0
        lax.fori_loop(0, CHUNK // 16, r_body, 0)
        pltpu.async_copy(rows[b], acc_sh.at[dstb[q]], sems[b], add=True)

    fill_scatter(0, 0, 0, False)
    fill_scatter(1, 1, 1, True)
    fill_scatter(2, 0, 2, True)
    fill_scatter(3, 1, 3, True)

    def m_body(m, _):
        k0 = 4 * m
        fill_scatter(k0, 0, 0, True)
        fill_scatter(k0 + 1, 1, 1, True)
        fill_scatter(k0 + 2, 0, 2, True)
        fill_scatter(k0 + 3, 1, 3, True)
        return 0
    lax.fori_loop(1, N_CHUNKS // 4, m_body, 0)

    for b in range(2):
        pltpu.make_async_copy(rows[b], acc_sh.at[dstb[0]], sems[b]).wait()
    for k in (N_CHUNKS, N_CHUNKS + 1):
        pltpu.make_async_copy(dst_hbm.at[w, k], dstb[k % 4], semd[k % 4]).wait()
        pltpu.make_async_copy(ew_hbm.at[w, k], ewb[k % 4], seme[k % 4]).wait()

    plsc.subcore_barrier()
    pltpu.sync_copy(acc_sh.at[pl.ds(row0, ROWS_PER_TILE)],
                    out_hbm.at[c, pl.ds(row0, ROWS_PER_TILE)])


def _agg_body(g_hbm, src_hbm, dst_hbm, ew_hbm, out_hbm, acc_sh, src_all,
              d0_v, d1_v, d2_v, e0_v, e1_v, e2_v, r0_v, r1_v, r2_v,
              g0_s, g1_s, g2_s, s0_s, s1_s, s2_s,
              sd0, sd1, sd2, se0, se1, se2):
    c = lax.axis_index("c")
    s = lax.axis_index("s")
    w = c * 16 + s
    rows = (r0_v, r1_v, r2_v)
    dstb = (d0_v, d1_v, d2_v)
    ewb = (e0_v, e1_v, e2_v)
    semg = (g0_s, g1_s, g2_s)
    sems = (s0_s, s1_s, s2_s)
    semd = (sd0, sd1, sd2)
    seme = (se0, se1, se2)

    pltpu.sync_copy(src_hbm.at[w], src_all)

    zeros16 = jnp.zeros((16,), _f32)

    def zb(i, _):
        for j in range(8):
            sl = pl.ds(j * 16, 16)
            r0_v[i, sl] = zeros16
            r1_v[i, sl] = zeros16
            r2_v[i, sl] = zeros16
        return 0
    lax.fori_loop(0, CHUNK, zb, 0)

    row0 = s * ROWS_PER_TILE
    for k in range(ROWS_PER_TILE // CHUNK):
        pltpu.sync_copy(r0_v, acc_sh.at[pl.ds(row0 + k * CHUNK, CHUNK)])
    plsc.subcore_barrier()

    descs = [pltpu.async_copy(rows[b], acc_sh.at[src_all.at[0]], sems[b],
                              add=True) for b in range(3)]
    for b in range(2):
        descs[b].wait()
        pltpu.async_copy(g_hbm.at[src_all.at[b]], rows[b], semg[b])
        pltpu.async_copy(dst_hbm.at[w, b], dstb[b], semd[b])
        pltpu.async_copy(ew_hbm.at[w, b], ewb[b], seme[b])

    def process(k, b):
        b2 = (b + 2) % 3
        pltpu.make_async_copy(g_hbm.at[src_all.at[k]], rows[b], semg[b]).wait()
        pltpu.make_async_copy(ew_hbm.at[w, k], ewb[b], seme[b]).wait()
        pltpu.make_async_copy(dst_hbm.at[w, k], dstb[b], semd[b]).wait()

        def r_body(i, _):
            wv = ewb[b][pl.ds(i * 16, 16)]
            for l in range(16):
                wb = jnp.full((16,), wv[l], dtype=_f32)
                r = i * 16 + l
                for j in range(8):
                    sl = pl.ds(j * 16, 16)
                    rows[b][r, sl] = rows[b][r, sl] * wb
            return 0
        lax.fori_loop(0, CHUNK // 16, r_body, 0)
        pltpu.async_copy(rows[b], acc_sh.at[dstb[b]], sems[b], add=True)
        pltpu.make_async_copy(rows[b2], acc_sh.at[src_all.at[0]], sems[b2]).wait()
        pltpu.async_copy(g_hbm.at[src_all.at[k + 2]], rows[b2], semg[b2])
        pltpu.async_copy(dst_hbm.at[w, k + 2], dstb[b2], semd[b2])
        pltpu.async_copy(ew_hbm.at[w, k + 2], ewb[b2], seme[b2])

    def m_body(m, _):
        k0 = 3 * m
        process(k0, 0)
        process(k0 + 1, 1)
        process(k0 + 2, 2)
        return 0
    _L = N_CHUNKS // 3
    lax.fori_loop(0, _L, m_body, 0)
    for k in range(3 * _L, N_CHUNKS):
        process(k, k % 3)

    pltpu.make_async_copy(rows[(N_CHUNKS - 1) % 3],
                          acc_sh.at[src_all.at[0]],
                          sems[(N_CHUNKS - 1) % 3]).wait()
    for k in (N_CHUNKS, N_CHUNKS + 1):
        pltpu.make_async_copy(g_hbm.at[src_all.at[k]],
                              rows[k % 3], semg[k % 3]).wait()
        pltpu.make_async_copy(dst_hbm.at[w, k], dstb[k % 3], semd[k % 3]).wait()
        pltpu.make_async_copy(ew_hbm.at[w, k], ewb[k % 3], seme[k % 3]).wait()

    plsc.subcore_barrier()
    pltpu.sync_copy(acc_sh.at[pl.ds(row0, ROWS_PER_TILE)],
                    out_hbm.at[c, pl.ds(row0, ROWS_PER_TILE)])


@functools.cache
def _sc_kernels():
    mesh = plsc.VectorSubcoreMesh(core_axis_name="c", subcore_axis_name="s")
    deg_k = pl.kernel(
        _deg_body,
        out_type=jax.ShapeDtypeStruct((2, NP, HID), _f32),
        mesh=mesh,
        scratch_types=(
            [pltpu.VMEM_SHARED((NP, HID), _f32)]
            + [pltpu.VMEM((CHUNK,), jnp.int32)] * 4
            + [pltpu.VMEM((CHUNK,), _f32)] * 4
            + [pltpu.VMEM((CHUNK, HID), _f32)] * 2
            + [pltpu.SemaphoreType.DMA] * 10
        ),
    )
    agg_k = pl.kernel(
        _agg_body,
        out_type=jax.ShapeDtypeStruct((2, NP, HID), _f32),
        mesh=mesh,
        scratch_types=(
            [pltpu.VMEM_SHARED((NP, HID), _f32),
             pltpu.VMEM((N_CHUNKS + 2, CHUNK), jnp.int32)]
            + [pltpu.VMEM((CHUNK,), jnp.int32)] * 3
            + [pltpu.VMEM((CHUNK,), _f32)] * 3
            + [pltpu.VMEM((CHUNK, HID), _f32)] * 3
            + [pltpu.SemaphoreType.DMA] * 12
        ),
    )
    return deg_k, agg_k


def kernel(node_features, edge_index, edge_attr, W_in, b_in, We1, be1, We2, be2,
           Wc0, bc0, Wc1, bc1, Wc2, bc2, Wo, bo):
    f32 = jnp.float32
    x = jnp.concatenate(
        [node_features.astype(f32), jnp.zeros((NP - N, IN_DIM), f32)])
    pad_idx = jnp.full((EPAD - E,), NP - 1, dtype=jnp.int32)
    src = jnp.concatenate([edge_index[0], pad_idx])
    dst = jnp.concatenate([edge_index[1], pad_idx])
    eaT = jnp.concatenate(
        [edge_attr.astype(f32), jnp.zeros((EPAD - E, EDGE_DIM), f32)]).T

    ew2 = pl.pallas_call(
        _edge_mlp_body,
        grid=(EPAD // EB,),
        in_specs=[
            pl.BlockSpec((EDGE_DIM, EB), lambda i: (0, i)),
            pl.BlockSpec((96, EDGE_DIM), lambda i: (0, 0)),
            pl.BlockSpec((96, 1), lambda i: (0, 0)),
            pl.BlockSpec((1, 96), lambda i: (0, 0)),
            pl.BlockSpec((1, 1), lambda i: (0, 0)),
        ],
        out_specs=pl.BlockSpec((1, EB), lambda i: (0, i)),
        out_shape=jax.ShapeDtypeStruct((1, EPAD), f32),
    )(eaT, We1.T, be1[:, None], We2.T, be2[:, None])
    pad2i = jnp.zeros((NW, 2, CHUNK), jnp.int32)
    pad2f = jnp.zeros((NW, 2, CHUNK), f32)
    srcP = jnp.concatenate([src.reshape(NW, N_CHUNKS, CHUNK), pad2i], axis=1)
    dstP = jnp.concatenate([dst.reshape(NW, N_CHUNKS, CHUNK), pad2i], axis=1)
    ewP = jnp.concatenate([ew2.reshape(NW, N_CHUNKS, CHUNK), pad2f], axis=1)

    deg_k, agg_k = _sc_kernels()

    degb = deg_k(dstP, ewP)

    h0, g0, dis_b = pl.pallas_call(
        _t0_body,
        grid=(_GRID,),
        in_specs=[_row_spec, _w_spec, _b_spec, _w_spec, _p_spec],
        out_specs=[_row_spec, _row_spec, _row_spec],
        out_shape=[_node_sds, _node_sds, _node_sds],
    )(x, W_in, b_in[None, :], Wc0, degb)

    h, g = h0, g0
    for W_next, b_prev, last in ((Wc1, bc0, False), (Wc2, bc1, False), (Wo, bc2, True)):
        p = agg_k(g, srcP, dstP, ewP)
        if not last:
            h, g = pl.pallas_call(
                _layer_body,
                grid=(_GRID,),
                in_specs=[_row_spec, _p_spec, _row_spec, _row_spec, _b_spec, _w_spec],
                out_specs=[_row_spec, _row_spec],
                out_shape=[_node_sds, _node_sds],
            )(h, p, g, dis_b, b_prev[None, :], W_next)
        else:
            out = pl.pallas_call(
                _final_body,
                grid=(_GRID,),
                in_specs=[_row_spec, _p_spec, _row_spec, _row_spec, _b_spec,
                          _w_spec, _b_spec],
                out_specs=_row_spec,
                out_shape=_node_sds,
            )(h, p, g, dis_b, b_prev[None, :], W_next, bo[None, :])
    return out[:N]


# speedup vs baseline: 8.8662x; 1.4377x over previous
import functools

import jax
import jax.numpy as jnp
from jax import lax
from jax.experimental import pallas as pl
from jax.experimental.pallas import tpu as pltpu
from jax.experimental.pallas import tpu_sc as plsc

N = 10000
IN_DIM = 128
EDGE_DIM = 16
HID = 128

NW = 32
NP = 10240
E = 320000
CHUNK = 64
N_CHUNKS = 162
PER_TILE = CHUNK * N_CHUNKS
EPAD = PER_TILE * NW
PADC = 4
ROWS_PER_TILE = NP // 16
EB = 1024

_f32 = jnp.float32
_bf16 = jnp.bfloat16

_PERM = []
for _j in range(4):
    for _t in range(16):
        _PERM.append(32 * _j + _t)
        _PERM.append(32 * _j + 16 + _t)


def _edge_mlp_body(eaT_ref, W1T_ref, b1_ref, W2T_ref, b2_ref, out_ref):
    t = jnp.dot(W1T_ref[...], eaT_ref[...], preferred_element_type=_f32)
    t = jnp.maximum(t + b1_ref[...], 0.0)
    s = jnp.dot(W2T_ref[...], t, preferred_element_type=_f32) + b2_ref[...]
    out_ref[...] = jax.nn.sigmoid(s)


def _t0_body(x_ref, Win_ref, bin_ref, Wc0_ref, degb_ref, h0_ref, g0_ref, dis_ref):
    db = degb_ref[...]
    d = lax.rsqrt(db[0] + db[1])
    h0 = jnp.dot(x_ref[...], Win_ref[...], preferred_element_type=_f32)
    h0 = jnp.maximum(h0 + bin_ref[...], 0.0)
    h0_ref[...] = h0
    dis_ref[...] = d
    g0_ref[...] = jnp.dot(h0, Wc0_ref[...], preferred_element_type=_f32) * d


def _layer_body(h_ref, p_ref, dis_ref, b_ref, W_ref, hn_ref, gn_ref):
    dis = dis_ref[...]
    hn = dis * (p_ref[0] + p_ref[1]) + b_ref[...] + h_ref[...]
    hn = jnp.maximum(hn, 0.0)
    hn_ref[...] = hn
    gn_ref[...] = jnp.dot(hn, W_ref[...], preferred_element_type=_f32) * dis


def _final_body(h_ref, p_ref, dis_ref, b_ref, Wo_ref, bo_ref, out_ref):
    dis = dis_ref[...]
    hn = dis * (p_ref[0] + p_ref[1]) + b_ref[...] + h_ref[...]
    hn = jnp.maximum(hn, 0.0)
    out_ref[...] = jnp.dot(hn, Wo_ref[...], preferred_element_type=_f32) + bo_ref[...]


_ROWB = 256
_GRID = NP // _ROWB

_row_spec = pl.BlockSpec((_ROWB, HID), lambda i: (i, 0))
_w_spec = pl.BlockSpec((HID, HID), lambda i: (0, 0))
_b_spec = pl.BlockSpec((1, HID), lambda i: (0, 0))
_p_spec = pl.BlockSpec((2, _ROWB, HID), lambda i: (0, i, 0))
_deg_spec = pl.BlockSpec((2, _ROWB, HID), lambda i: (0, i, 0))
_node_sds = jax.ShapeDtypeStruct((NP, HID), _f32)
_node_sds_bf = jax.ShapeDtypeStruct((NP, HID), _bf16)


def _deg_body(dst_hbm, ew_hbm, out_hbm, acc_sh,
              d0_v, d1_v, d2_v, d3_v, e0_v, e1_v, e2_v, e3_v, r0_v, r1_v,
              s0_s, s1_s, sd0, sd1, sd2, sd3, se0, se1, se2, se3):
    c = lax.axis_index("c")
    s = lax.axis_index("s")
    w = c * 16 + s
    rows = (r0_v, r1_v)
    dstb = (d0_v, d1_v, d2_v, d3_v)
    ewb = (e0_v, e1_v, e2_v, e3_v)
    sems = (s0_s, s1_s)
    semd = (sd0, sd1, sd2, sd3)
    seme = (se0, se1, se2, se3)

    pltpu.sync_copy(dst_hbm.at[w, 0], dstb[0])
    pltpu.sync_copy(ew_hbm.at[w, 0], ewb[0])
    pltpu.async_copy(dst_hbm.at[w, 1], dstb[1], semd[1])
    pltpu.async_copy(ew_hbm.at[w, 1], ewb[1], seme[1])

    zeros16 = jnp.zeros((16,), _f32)

    def zb(i, _):
        for j in range(8):
            sl = pl.ds(j * 16, 16)
            r0_v[i, sl] = zeros16
            r1_v[i, sl] = zeros16
        return 0
    lax.fori_loop(0, CHUNK, zb, 0)

    row0 = s * ROWS_PER_TILE
    for k in range(ROWS_PER_TILE // CHUNK):
        pltpu.sync_copy(r0_v, acc_sh.at[pl.ds(row0 + k * CHUNK, CHUNK)])
    plsc.subcore_barrier()

    for b in range(2):
        pltpu.async_copy(rows[b], acc_sh.at[dstb[0]], sems[b], add=True)

    def fill_scatter(k, b, q, wait_idx):
        q2 = (q + 2) % 4
        pltpu.make_async_copy(rows[b], acc_sh.at[dstb[0]], sems[b]).wait()
        pltpu.async_copy(dst_hbm.at[w, k + 2], dstb[q2], semd[q2])
        pltpu.async_copy(ew_hbm.at[w, k + 2], ewb[q2], seme[q2])
        if wait_idx:
            pltpu.make_async_copy(ew_hbm.at[w, k], ewb[q], seme[q]).wait()
            pltpu.make_async_copy(dst_hbm.at[w, k], dstb[q], semd[q]).wait()

        def r_body(i, _):
            wv = ewb[q][pl.ds(i * 16, 16)]
            for l in range(16):
                wb = jnp.full((16,), wv[l], dtype=_f32)
                for j in range(8):
                    rows[b][i * 16 + l, pl.ds(j * 16, 16)] = wb
            return 0
        lax.fori_loop(0, CHUNK // 16, r_body, 0)
        pltpu.async_copy(rows[b], acc_sh.at[dstb[q]], sems[b], add=True)

    fill_scatter(0, 0, 0, False)
    fill_scatter(1, 1, 1, True)
    fill_scatter(2, 0, 2, True)
    fill_scatter(3, 1, 3, True)

    def m_body(m, _):
        k0 = 4 * m
        fill_scatter(k0, 0, 0, True)
        fill_scatter(k0 + 1, 1, 1, True)
        fill_scatter(k0 + 2, 0, 2, True)
        fill_scatter(k0 + 3, 1, 3, True)
        return 0
    lax.fori_loop(1, N_CHUNKS // 4, m_body, 0)
    for k in range(4 * (N_CHUNKS // 4), N_CHUNKS):
        fill_scatter(k, k % 2, k % 4, True)

    for b in range(2):
        pltpu.make_async_copy(rows[b], acc_sh.at[dstb[0]], sems[b]).wait()
    for k in (N_CHUNKS, N_CHUNKS + 1):
        pltpu.make_async_copy(dst_hbm.at[w, k], dstb[k % 4], semd[k % 4]).wait()
        pltpu.make_async_copy(ew_hbm.at[w, k], ewb[k % 4], seme[k % 4]).wait()

    plsc.subcore_barrier()
    pltpu.sync_copy(acc_sh.at[pl.ds(row0, ROWS_PER_TILE)],
                    out_hbm.at[c, pl.ds(row0, ROWS_PER_TILE)])


def _agg_body(g_hbm, src_hbm, dst_hbm, ew_hbm, out_hbm, acc_sh, src_all,
              d0_v, d1_v, d2_v, e0_v, e1_v, e2_v,
              r0_v, r1_v, r2_v,
              g0_s, g1_s, g2_s, sd0, sd1, sd2,
              se0, se1, se2, s0_s, s1_s, s2_s):
    c = lax.axis_index("c")
    s = lax.axis_index("s")
    w = c * 16 + s
    dstb = (d0_v, d1_v, d2_v)
    ewb = (e0_v, e1_v, e2_v)
    rows = (r0_v, r1_v, r2_v)
    semg = (g0_s, g1_s, g2_s)
    semd = (sd0, sd1, sd2)
    seme = (se0, se1, se2)
    sems = (s0_s, s1_s, s2_s)

    pltpu.sync_copy(src_hbm.at[w], src_all)

    zeros16 = jnp.zeros((16,), _f32)

    def zb(i, _):
        for j in range(8):
            sl = pl.ds(j * 16, 16)
            r0_v[i, sl] = zeros16
            r1_v[i, sl] = zeros16
            r2_v[i, sl] = zeros16
        return 0
    lax.fori_loop(0, CHUNK, zb, 0)

    row0 = s * ROWS_PER_TILE
    for k in range(ROWS_PER_TILE // CHUNK):
        pltpu.sync_copy(r0_v, acc_sh.at[pl.ds(row0 + k * CHUNK, CHUNK)])
    plsc.subcore_barrier()

    descs = [pltpu.async_copy(rows[b], acc_sh.at[src_all.at[0]], sems[b],
                              add=True) for b in range(3)]
    for b in range(2):
        descs[b].wait()
        pltpu.async_copy(g_hbm.at[src_all.at[b]], rows[b], semg[b])
        pltpu.async_copy(dst_hbm.at[w, b], dstb[b], semd[b])
        pltpu.async_copy(ew_hbm.at[w, b], ewb[b], seme[b])

    def process(k, b):
        b2 = (b + 2) % 3
        pltpu.make_async_copy(g_hbm.at[src_all.at[k]], rows[b], semg[b]).wait()
        pltpu.make_async_copy(ew_hbm.at[w, k], ewb[b], seme[b]).wait()
        pltpu.make_async_copy(dst_hbm.at[w, k], dstb[b], semd[b]).wait()

        def r_body(i, _):
            wv = ewb[b][pl.ds(i * 16, 16)]
            for l in range(16):
                wb = jnp.full((16,), wv[l], dtype=_f32)
                r = i * 16 + l
                for j in range(8):
                    sl = pl.ds(j * 16, 16)
                    rows[b][r, sl] = rows[b][r, sl] * wb
            return 0
        lax.fori_loop(0, CHUNK // 16, r_body, 0)
        pltpu.async_copy(rows[b], acc_sh.at[dstb[b]], sems[b], add=True)
        pltpu.make_async_copy(rows[b2], acc_sh.at[src_all.at[0]], sems[b2]).wait()
        pltpu.async_copy(g_hbm.at[src_all.at[k + 2]], rows[b2], semg[b2])
        pltpu.async_copy(dst_hbm.at[w, k + 2], dstb[b2], semd[b2])
        pltpu.async_copy(ew_hbm.at[w, k + 2], ewb[b2], seme[b2])

    def m_body(m, _):
        k0 = 3 * m
        process(k0, 0)
        process(k0 + 1, 1)
        process(k0 + 2, 2)
        return 0
    lax.fori_loop(0, N_CHUNKS // 3, m_body, 0)

    pltpu.make_async_copy(rows[(N_CHUNKS - 1) % 3], acc_sh.at[src_all.at[0]],
                          sems[(N_CHUNKS - 1) % 3]).wait()
    for k in (N_CHUNKS, N_CHUNKS + 1):
        pltpu.make_async_copy(g_hbm.at[src_all.at[k]],
                              rows[k % 3], semg[k % 3]).wait()
        pltpu.make_async_copy(dst_hbm.at[w, k], dstb[k % 3], semd[k % 3]).wait()
        pltpu.make_async_copy(ew_hbm.at[w, k], ewb[k % 3], seme[k % 3]).wait()

    plsc.subcore_barrier()
    pltpu.sync_copy(acc_sh.at[pl.ds(row0, ROWS_PER_TILE)],
                    out_hbm.at[c, pl.ds(row0, ROWS_PER_TILE)])


@functools.cache
def _sc_kernels():
    mesh = plsc.VectorSubcoreMesh(core_axis_name="c", subcore_axis_name="s")
    deg_k = pl.kernel(
        _deg_body,
        out_type=jax.ShapeDtypeStruct((2, NP, HID), _f32),
        mesh=mesh,
        scratch_types=(
            [pltpu.VMEM_SHARED((NP, HID), _f32)]
            + [pltpu.VMEM((CHUNK,), jnp.int32)] * 4
            + [pltpu.VMEM((CHUNK,), _f32)] * 4
            + [pltpu.VMEM((CHUNK, HID), _f32)] * 2
            + [pltpu.SemaphoreType.DMA] * 10
        ),
    )
    agg_k = pl.kernel(
        _agg_body,
        out_type=jax.ShapeDtypeStruct((2, NP, HID), _f32),
        mesh=mesh,
        scratch_types=(
            [pltpu.VMEM_SHARED((NP, HID), _f32),
             pltpu.VMEM((N_CHUNKS + PADC, CHUNK), jnp.int32)]
            + [pltpu.VMEM((CHUNK,), jnp.int32)] * 3
            + [pltpu.VMEM((CHUNK,), _f32)] * 3
            + [pltpu.VMEM((CHUNK, HID), _f32)] * 3
            + [pltpu.SemaphoreType.DMA] * 12
        ),
    )
    return deg_k, agg_k


def kernel(node_features, edge_index, edge_attr, W_in, b_in, We1, be1, We2, be2,
           Wc0, bc0, Wc1, bc1, Wc2, bc2, Wo, bo):
    f32 = jnp.float32
    x = jnp.concatenate(
        [node_features.astype(f32), jnp.zeros((NP - N, IN_DIM), f32)])
    loop_idx = jnp.arange(NP, dtype=jnp.int32)
    n_fill = EPAD - E - NP
    pad_idx = jnp.full((n_fill,), NP - 1, dtype=jnp.int32)
    src = jnp.concatenate([edge_index[0], loop_idx, pad_idx])
    dst = jnp.concatenate([edge_index[1], loop_idx, pad_idx])
    eaT = jnp.concatenate(
        [edge_attr.astype(f32), jnp.zeros((EPAD - E, EDGE_DIM), f32)]).T

    ew2 = pl.pallas_call(
        _edge_mlp_body,
        grid=(EPAD // EB,),
        in_specs=[
            pl.BlockSpec((EDGE_DIM, EB), lambda i: (0, i)),
            pl.BlockSpec((96, EDGE_DIM), lambda i: (0, 0)),
            pl.BlockSpec((96, 1), lambda i: (0, 0)),
            pl.BlockSpec((1, 96), lambda i: (0, 0)),
            pl.BlockSpec((1, 1), lambda i: (0, 0)),
        ],
        out_specs=pl.BlockSpec((1, EB), lambda i: (0, i)),
        out_shape=jax.ShapeDtypeStruct((1, EPAD), f32),
    )(eaT, We1.T, be1[:, None], We2.T, be2[:, None])
    ew = jnp.concatenate(
        [ew2.reshape(EPAD)[:E], jnp.ones((NP,), f32), jnp.zeros((n_fill,), f32)])

    padi = jnp.zeros((NW, PADC, CHUNK), jnp.int32)
    padf = jnp.zeros((NW, PADC, CHUNK), f32)
    srcP = jnp.concatenate([src.reshape(NW, N_CHUNKS, CHUNK), padi], axis=1)
    dstP = jnp.concatenate([dst.reshape(NW, N_CHUNKS, CHUNK), padi], axis=1)
    ewP = jnp.concatenate([ew.reshape(NW, N_CHUNKS, CHUNK), padf], axis=1)

    deg_k, agg_k = _sc_kernels()

    degb = deg_k(dstP, ewP)

    h0, g0, dis_b = pl.pallas_call(
        _t0_body,
        grid=(_GRID,),
        in_specs=[_row_spec, _w_spec, _b_spec, _w_spec, _deg_spec],
        out_specs=[_row_spec, _row_spec, _row_spec],
        out_shape=[_node_sds, _node_sds, _node_sds],
    )(x, W_in, b_in[None, :], Wc0, degb)

    h, g = h0, g0
    for W_next, b_prev, last in ((Wc1, bc0, False), (Wc2, bc1, False), (Wo, bc2, True)):
        p = agg_k(g, srcP, dstP, ewP)
        if not last:
            h, g = pl.pallas_call(
                _layer_body,
                grid=(_GRID,),
                in_specs=[_row_spec, _p_spec, _row_spec, _b_spec, _w_spec],
                out_specs=[_row_spec, _row_spec],
                out_shape=[_node_sds, _node_sds],
            )(h, p, dis_b, b_prev[None, :], W_next)
        else:
            out = pl.pallas_call(
                _final_body,
                grid=(_GRID,),
                in_specs=[_row_spec, _p_spec, _row_spec, _b_spec,
                          _w_spec, _b_spec],
                out_specs=_row_spec,
                out_shape=_node_sds,
            )(h, p, dis_b, b_prev[None, :], W_next, bo[None, :])
    return out[:N]
